# baseline (speedup 1.0000x reference)
import functools

import jax
import jax.numpy as jnp
from jax import lax
from jax.experimental import pallas as pl
from jax.experimental.pallas import tpu as pltpu
from jax.experimental.pallas import tpu_sc as plsc

_N = 1024
_L = 16
_NC = 2
_NS = 16


def _sc_auction_body(eps_hbm, it_hbm, x1a, x1b, x1c, x2a, x2b, x2c,
                     ass_hbm, dist_hbm,
                     par_v, x1p0, x1p1, x1p2, x2p0, x2p1, x2p2,
                     price_v, ainv_v, unass_v, maxinc_v, winner_v,
                     ass_v, dist_v):
    n = _N
    nb = ass_hbm.shape[0]
    wid = lax.axis_index("s") * _NC + lax.axis_index("c")
    lane = lax.broadcasted_iota(jnp.int32, (_L,), 0)
    ninf = jnp.float32(-jnp.inf)
    nblk = n // _L

    def one_batch(b):
        pltpu.sync_copy(eps_hbm, par_v)
        eps = par_v[pl.ds(0, _L)][0]
        pltpu.sync_copy(it_hbm, par_v)
        iters = par_v[pl.ds(0, _L)][0].astype(jnp.int32)
        pltpu.sync_copy(x1a.at[b], x1p0)
        pltpu.sync_copy(x1b.at[b], x1p1)
        pltpu.sync_copy(x1c.at[b], x1p2)
        pltpu.sync_copy(x2a.at[b], x2p0)
        pltpu.sync_copy(x2b.at[b], x2p1)
        pltpu.sync_copy(x2c.at[b], x2p2)

        def init_blk(ib, c):
            sl = pl.ds(ib * _L, _L)
            price_v[sl] = jnp.zeros((_L,), jnp.float32)
            ainv_v[sl] = jnp.full((_L,), -1, jnp.int32)
            unass_v[sl] = jnp.full((_L,), 1, jnp.int32)
            maxinc_v[sl] = jnp.full((_L,), ninf, jnp.float32)
            winner_v[sl] = jnp.full((_L,), n, jnp.int32)
            ass_v[sl] = jnp.full((_L,), -1, jnp.int32)
            dist_v[sl] = jnp.zeros((_L,), jnp.float32)
            return c

        lax.fori_loop(0, nblk, init_blk, 0)

        def scan_row(i):
            isp = jnp.full((_L,), i, jnp.int32)
            x1s0 = plsc.load_gather(x1p0, [isp])
            x1s1 = plsc.load_gather(x1p1, [isp])
            x1s2 = plsc.load_gather(x1p2, [isp])

            def chunk(ch, carry):
                m1, m2, i1 = carry
                sl = pl.ds(ch * _L, _L)
                d0 = x1s0 - x2p0[sl]
                d1 = x1s1 - x2p1[sl]
                d2 = x1s2 - x2p2[sl]
                w = ((d0 * d0 + d1 * d1) + d2 * d2) + price_v[sl]
                idx = ch * _L + lane
                lt = w < m1
                lt2 = w < m2
                m2n = jnp.where(lt, m1, jnp.where(lt2, w, m2))
                i1n = jnp.where(lt, idx, i1)
                m1n = jnp.where(lt, w, m1)
                return m1n, m2n, i1n

            inf16 = jnp.full((_L,), jnp.inf, jnp.float32)
            m1, m2, i1 = lax.fori_loop(
                0, nblk, chunk, (inf16, inf16, jnp.zeros((_L,), jnp.int32)),
                unroll=4)
            best = jnp.min(m1)
            isb = m1 == best
            cnt = jnp.sum(jnp.where(isb, 1, 0))
            bidx = jnp.min(jnp.where(isb, i1, n))
            sec0 = jnp.min(jnp.where(isb, m2, m1))
            secs = jnp.where(cnt > 1, best, sec0)
            binc = (secs - best) + eps

            bsp = jnp.full((_L,), bidx, jnp.int32)
            cur = plsc.load_gather(maxinc_v, [bsp])
            curw = plsc.load_gather(winner_v, [bsp])
            bincs = jnp.full((_L,), binc)
            better = bincs > cur
            tie = bincs == cur
            neww = jnp.where(better, isp,
                             jnp.where(tie, jnp.minimum(curw, isp), curw))
            plsc.store_scatter(maxinc_v, [bsp], jnp.maximum(cur, bincs),
                               mask=lane == 0)
            plsc.store_scatter(winner_v, [bsp], neww, mask=lane == 0)

        def phase_a(ib, c):
            flags = unass_v[pl.ds(ib * _L, _L)]

            @pl.when(jnp.max(flags) > 0)
            def _():
                for r in range(_L):
                    @pl.when(flags[r] > 0)
                    def _():
                        scan_row(ib * _L + r)

            return c

        def phase_b(ch, c):
            sl = pl.ds(ch * _L, _L)
            mi = maxinc_v[sl]
            hb = mi > ninf
            win = winner_v[sl]
            prev = ainv_v[sl]
            price_v[sl] = jnp.where(hb, price_v[sl] + mi, price_v[sl])
            ainv_v[sl] = jnp.where(hb, win, prev)
            mprev = hb & (prev >= 0)
            plsc.store_scatter(unass_v, [jnp.maximum(prev, 0)],
                               jnp.full((_L,), 1, jnp.int32), mask=mprev)
            plsc.store_scatter(unass_v, [jnp.where(hb, win, 0)],
                               jnp.zeros((_L,), jnp.int32), mask=hb)
            maxinc_v[sl] = jnp.full((_L,), ninf, jnp.float32)
            winner_v[sl] = jnp.full((_L,), n, jnp.int32)
            return c

        def round_body(t, c):
            lax.fori_loop(0, nblk, phase_a, 0)
            lax.fori_loop(0, nblk, phase_b, 0)
            return c

        lax.fori_loop(0, iters, round_body, 0)

        def epi(ch, c):
            sl = pl.ds(ch * _L, _L)
            jids = ch * _L + lane
            owners = ainv_v[sl]
            m = owners >= 0
            oc = jnp.maximum(owners, 0)
            d0 = plsc.load_gather(x1p0, [oc]) - x2p0[sl]
            d1 = plsc.load_gather(x1p1, [oc]) - x2p1[sl]
            d2 = plsc.load_gather(x1p2, [oc]) - x2p2[sl]
            dd = (d0 * d0 + d1 * d1) + d2 * d2
            plsc.store_scatter(dist_v, [oc], dd, mask=m)
            plsc.store_scatter(ass_v, [oc], jids, mask=m)
            return c

        lax.fori_loop(0, nblk, epi, 0)
        pltpu.sync_copy(ass_v, ass_hbm.at[b])
        pltpu.sync_copy(dist_v, dist_hbm.at[b])

    @pl.when(wid < nb)
    def _():
        one_batch(wid)


def kernel(input1, input2, eps, iters):
    b, n, _ = input1.shape
    x1planes = [input1[:, :, k] for k in range(3)]
    x2planes = [input2[:, :, k] for k in range(3)]
    eps_a = jnp.full((_L,), eps, jnp.float32)
    it_a = jnp.full((_L,), iters, jnp.float32)
    mesh = plsc.VectorSubcoreMesh(
        core_axis_name="c", subcore_axis_name="s",
        num_cores=_NC, num_subcores=_NS)
    f = functools.partial(
        pl.kernel,
        out_type=[
            jax.ShapeDtypeStruct((b, _N), jnp.int32),
            jax.ShapeDtypeStruct((b, _N), jnp.float32),
        ],
        mesh=mesh,
        compiler_params=pltpu.CompilerParams(
            use_tc_tiling_on_sc=False, needs_layout_passes=False),
        scratch_types=(
            [pltpu.VMEM((_L,), jnp.float32)]
            + [pltpu.VMEM((_N,), jnp.float32)] * 6
            + [pltpu.VMEM((_N,), jnp.float32),
               pltpu.VMEM((_N,), jnp.int32),
               pltpu.VMEM((_N,), jnp.int32),
               pltpu.VMEM((_N,), jnp.float32),
               pltpu.VMEM((_N,), jnp.int32),
               pltpu.VMEM((_N,), jnp.int32),
               pltpu.VMEM((_N,), jnp.float32)]
        ),
    )(_sc_auction_body)
    ass, dist = f(eps_a, it_a, *x1planes, *x2planes)
    return dist, ass


# speedup vs baseline: 2.0486x; 2.0486x over previous
import functools

import jax
import jax.numpy as jnp
from jax import lax
from jax.experimental import pallas as pl
from jax.experimental.pallas import tpu as pltpu
from jax.experimental.pallas import tpu_sc as plsc

_N = 1024
_L = 16
_NC = 2
_NS = 16


def _sc_auction_body(eps_hbm, it_hbm, x1a, x1b, x1c, x2a, x2b, x2c,
                     ass_hbm, dist_hbm,
                     par_v, x1p0, x1p1, x1p2, x2p0, x2p1, x2p2,
                     price_v, ainv_v, unass_v, maxinc_v, winner_v,
                     ass_v, dist_v, pmax_v, pwin_v, sh_max, sh_win):
    n = _N
    sc = lax.axis_index("c")
    s = lax.axis_index("s")
    b = sc * (_NS // 2) + lax.rem(s, _NS // 2)
    half = s // (_NS // 2)
    partner = jnp.bitwise_xor(s, _NS // 2)
    lane = lax.broadcasted_iota(jnp.int32, (_L,), 0)
    ninf = jnp.float32(-jnp.inf)
    nblk = n // _L
    hblk = nblk // 2

    pltpu.sync_copy(eps_hbm, par_v)
    eps = par_v[pl.ds(0, _L)][0]
    pltpu.sync_copy(it_hbm, par_v)
    iters = par_v[pl.ds(0, _L)][0].astype(jnp.int32)
    pltpu.sync_copy(x1a.at[b], x1p0)
    pltpu.sync_copy(x1b.at[b], x1p1)
    pltpu.sync_copy(x1c.at[b], x1p2)
    pltpu.sync_copy(x2a.at[b], x2p0)
    pltpu.sync_copy(x2b.at[b], x2p1)
    pltpu.sync_copy(x2c.at[b], x2p2)

    def init_blk(ib, c):
        sl = pl.ds(ib * _L, _L)
        price_v[sl] = jnp.zeros((_L,), jnp.float32)
        ainv_v[sl] = jnp.full((_L,), -1, jnp.int32)
        unass_v[sl] = jnp.full((_L,), 1, jnp.int32)
        maxinc_v[sl] = jnp.full((_L,), ninf, jnp.float32)
        winner_v[sl] = jnp.full((_L,), n, jnp.int32)
        ass_v[sl] = jnp.full((_L,), -1, jnp.int32)
        dist_v[sl] = jnp.zeros((_L,), jnp.float32)
        return c

    lax.fori_loop(0, nblk, init_blk, 0)

    def scan_row(i):
        isp = jnp.full((_L,), i, jnp.int32)
        x1s0 = plsc.load_gather(x1p0, [isp])
        x1s1 = plsc.load_gather(x1p1, [isp])
        x1s2 = plsc.load_gather(x1p2, [isp])

        def chunk(ch, carry):
            m1, m2, i1 = carry
            sl = pl.ds(ch * _L, _L)
            d0 = x1s0 - x2p0[sl]
            d1 = x1s1 - x2p1[sl]
            d2 = x1s2 - x2p2[sl]
            w = ((d0 * d0 + d1 * d1) + d2 * d2) + price_v[sl]
            idx = ch * _L + lane
            lt = w < m1
            lt2 = w < m2
            m2n = jnp.where(lt, m1, jnp.where(lt2, w, m2))
            i1n = jnp.where(lt, idx, i1)
            m1n = jnp.where(lt, w, m1)
            return m1n, m2n, i1n

        inf16 = jnp.full((_L,), jnp.inf, jnp.float32)
        m1, m2, i1 = lax.fori_loop(
            0, nblk, chunk, (inf16, inf16, jnp.zeros((_L,), jnp.int32)))
        best = jnp.min(m1)
        isb = m1 == best
        cnt = jnp.sum(jnp.where(isb, 1, 0))
        bidx = jnp.min(jnp.where(isb, i1, n))
        sec0 = jnp.min(jnp.where(isb, m2, m1))
        secs = jnp.where(cnt > 1, best, sec0)
        binc = (secs - best) + eps

        bsp = jnp.full((_L,), bidx, jnp.int32)
        cur = plsc.load_gather(maxinc_v, [bsp])
        curw = plsc.load_gather(winner_v, [bsp])
        bincs = jnp.full((_L,), binc)
        better = bincs > cur
        tie = bincs == cur
        neww = jnp.where(better, isp,
                         jnp.where(tie, jnp.minimum(curw, isp), curw))
        plsc.store_scatter(maxinc_v, [bsp], jnp.maximum(cur, bincs),
                           mask=lane == 0)
        plsc.store_scatter(winner_v, [bsp], neww, mask=lane == 0)

    def phase_a(ib, c):
        flags = unass_v[pl.ds(ib * _L, _L)]

        @pl.when(jnp.max(flags) > 0)
        def _():
            for r in range(_L):
                @pl.when(flags[r] > 0)
                def _():
                    scan_row(ib * _L + r)

        return c

    def phase_b(ch, c):
        sl = pl.ds(ch * _L, _L)
        a = maxinc_v[sl]
        pm = pmax_v[sl]
        wa = winner_v[sl]
        wp = pwin_v[sl]
        mi = jnp.maximum(a, pm)
        win = jnp.where(a > pm, wa,
                        jnp.where(pm > a, wp, jnp.minimum(wa, wp)))
        hb = mi > ninf
        prev = ainv_v[sl]
        price_v[sl] = jnp.where(hb, price_v[sl] + mi, price_v[sl])
        ainv_v[sl] = jnp.where(hb, win, prev)
        mprev = hb & (prev >= 0)
        plsc.store_scatter(unass_v, [jnp.maximum(prev, 0)],
                           jnp.full((_L,), 1, jnp.int32), mask=mprev)
        plsc.store_scatter(unass_v, [jnp.where(hb, win, 0)],
                           jnp.zeros((_L,), jnp.int32), mask=hb)
        maxinc_v[sl] = jnp.full((_L,), ninf, jnp.float32)
        winner_v[sl] = jnp.full((_L,), n, jnp.int32)
        return c

    def round_body(t, c):
        lax.fori_loop(half * hblk, half * hblk + hblk, phase_a, 0)
        pltpu.sync_copy(maxinc_v, sh_max.at[s])
        pltpu.sync_copy(winner_v, sh_win.at[s])
        plsc.subcore_barrier()
        pltpu.sync_copy(sh_max.at[partner], pmax_v)
        pltpu.sync_copy(sh_win.at[partner], pwin_v)
        plsc.subcore_barrier()
        lax.fori_loop(0, nblk, phase_b, 0)
        return c

    lax.fori_loop(0, iters, round_body, 0)

    def epi(ch, c):
        sl = pl.ds(ch * _L, _L)
        jids = ch * _L + lane
        owners = ainv_v[sl]
        m = owners >= 0
        oc = jnp.maximum(owners, 0)
        d0 = plsc.load_gather(x1p0, [oc]) - x2p0[sl]
        d1 = plsc.load_gather(x1p1, [oc]) - x2p1[sl]
        d2 = plsc.load_gather(x1p2, [oc]) - x2p2[sl]
        dd = (d0 * d0 + d1 * d1) + d2 * d2
        plsc.store_scatter(dist_v, [oc], dd, mask=m)
        plsc.store_scatter(ass_v, [oc], jids, mask=m)
        return c

    @pl.when(half == 0)
    def _():
        lax.fori_loop(0, nblk, epi, 0)
        pltpu.sync_copy(ass_v, ass_hbm.at[b])
        pltpu.sync_copy(dist_v, dist_hbm.at[b])


def kernel(input1, input2, eps, iters):
    b, n, _ = input1.shape
    x1planes = [input1[:, :, k] for k in range(3)]
    x2planes = [input2[:, :, k] for k in range(3)]
    eps_a = jnp.full((_L,), eps, jnp.float32)
    it_a = jnp.full((_L,), iters, jnp.float32)
    mesh = plsc.VectorSubcoreMesh(
        core_axis_name="c", subcore_axis_name="s",
        num_cores=_NC, num_subcores=_NS)
    f = functools.partial(
        pl.kernel,
        out_type=[
            jax.ShapeDtypeStruct((b, _N), jnp.int32),
            jax.ShapeDtypeStruct((b, _N), jnp.float32),
        ],
        mesh=mesh,
        compiler_params=pltpu.CompilerParams(
            use_tc_tiling_on_sc=False, needs_layout_passes=False),
        scratch_types=(
            [pltpu.VMEM((_L,), jnp.float32)]
            + [pltpu.VMEM((_N,), jnp.float32)] * 6
            + [pltpu.VMEM((_N,), jnp.float32),
               pltpu.VMEM((_N,), jnp.int32),
               pltpu.VMEM((_N,), jnp.int32),
               pltpu.VMEM((_N,), jnp.float32),
               pltpu.VMEM((_N,), jnp.int32),
               pltpu.VMEM((_N,), jnp.int32),
               pltpu.VMEM((_N,), jnp.float32),
               pltpu.VMEM((_N,), jnp.float32),
               pltpu.VMEM((_N,), jnp.int32),
               pltpu.VMEM_SHARED((_NS, _N), jnp.float32),
               pltpu.VMEM_SHARED((_NS, _N), jnp.int32)]
        ),
    )(_sc_auction_body)
    ass, dist = f(eps_a, it_a, *x1planes, *x2planes)
    return dist, ass


# speedup vs baseline: 2.1403x; 1.0447x over previous
import functools

import jax
import jax.numpy as jnp
from jax import lax
from jax.experimental import pallas as pl
from jax.experimental.pallas import tpu as pltpu
from jax.experimental.pallas import tpu_sc as plsc

_N = 1024
_L = 16
_NC = 2
_NS = 16


def _sc_auction_body(eps_hbm, it_hbm, x1a, x1b, x1c, x2a, x2b, x2c,
                     ass_hbm, dist_hbm,
                     par_v, x1p0, x1p1, x1p2, x2p0, x2p1, x2p2,
                     price_v, ainv_v, unass_v, maxinc_v, winner_v,
                     ass_v, dist_v, pmax_v, pwin_v, sh_max, sh_win):
    n = _N
    sc = lax.axis_index("c")
    s = lax.axis_index("s")
    b = sc * (_NS // 2) + lax.rem(s, _NS // 2)
    half = s // (_NS // 2)
    partner = jnp.bitwise_xor(s, _NS // 2)
    lane = lax.broadcasted_iota(jnp.int32, (_L,), 0)
    ninf = jnp.float32(-jnp.inf)
    nblk = n // _L
    hblk = nblk // 2

    pltpu.sync_copy(eps_hbm, par_v)
    eps = par_v[pl.ds(0, _L)][0]
    pltpu.sync_copy(it_hbm, par_v)
    iters = par_v[pl.ds(0, _L)][0].astype(jnp.int32)
    pltpu.sync_copy(x1a.at[b], x1p0)
    pltpu.sync_copy(x1b.at[b], x1p1)
    pltpu.sync_copy(x1c.at[b], x1p2)
    pltpu.sync_copy(x2a.at[b], x2p0)
    pltpu.sync_copy(x2b.at[b], x2p1)
    pltpu.sync_copy(x2c.at[b], x2p2)

    def init_blk(ib, c):
        sl = pl.ds(ib * _L, _L)
        price_v[sl] = jnp.zeros((_L,), jnp.float32)
        ainv_v[sl] = jnp.full((_L,), -1, jnp.int32)
        unass_v[sl] = jnp.full((_L,), 1, jnp.int32)
        maxinc_v[sl] = jnp.full((_L,), ninf, jnp.float32)
        winner_v[sl] = jnp.full((_L,), n, jnp.int32)
        ass_v[sl] = jnp.full((_L,), -1, jnp.int32)
        dist_v[sl] = jnp.zeros((_L,), jnp.float32)
        return c

    lax.fori_loop(0, nblk, init_blk, 0)

    def scan_row(i):
        isp = jnp.full((_L,), i, jnp.int32)
        x1s0 = plsc.load_gather(x1p0, [isp])
        x1s1 = plsc.load_gather(x1p1, [isp])
        x1s2 = plsc.load_gather(x1p2, [isp])

        def chunk(ch, carry):
            m1, m2, i1 = carry
            sl = pl.ds(ch * _L, _L)
            d0 = x1s0 - x2p0[sl]
            d1 = x1s1 - x2p1[sl]
            d2 = x1s2 - x2p2[sl]
            w = ((d0 * d0 + d1 * d1) + d2 * d2) + price_v[sl]
            idx = ch * _L + lane
            lt = w < m1
            lt2 = w < m2
            m2n = jnp.where(lt, m1, jnp.where(lt2, w, m2))
            i1n = jnp.where(lt, idx, i1)
            m1n = jnp.where(lt, w, m1)
            return m1n, m2n, i1n

        inf16 = jnp.full((_L,), jnp.inf, jnp.float32)
        m1, m2, i1 = lax.fori_loop(
            0, nblk, chunk, (inf16, inf16, jnp.zeros((_L,), jnp.int32)),
            unroll=2)
        best = jnp.min(m1)
        isb = m1 == best
        cnt = jnp.sum(jnp.where(isb, 1, 0))
        bidx = jnp.min(jnp.where(isb, i1, n))
        sec0 = jnp.min(jnp.where(isb, m2, m1))
        secs = jnp.where(cnt > 1, best, sec0)
        binc = (secs - best) + eps

        bsp = jnp.full((_L,), bidx, jnp.int32)
        cur = plsc.load_gather(maxinc_v, [bsp])
        curw = plsc.load_gather(winner_v, [bsp])
        bincs = jnp.full((_L,), binc)
        better = bincs > cur
        tie = bincs == cur
        neww = jnp.where(better, isp,
                         jnp.where(tie, jnp.minimum(curw, isp), curw))
        plsc.store_scatter(maxinc_v, [bsp], jnp.maximum(cur, bincs),
                           mask=lane == 0)
        plsc.store_scatter(winner_v, [bsp], neww, mask=lane == 0)

    def phase_a(ib, c):
        flags = unass_v[pl.ds(ib * _L, _L)]

        @pl.when(jnp.max(flags) > 0)
        def _():
            for r in range(_L):
                @pl.when(flags[r] > 0)
                def _():
                    scan_row(ib * _L + r)

        return c

    def phase_b(ch, c):
        sl = pl.ds(ch * _L, _L)
        a = maxinc_v[sl]
        pm = pmax_v[sl]
        wa = winner_v[sl]
        wp = pwin_v[sl]
        mi = jnp.maximum(a, pm)
        win = jnp.where(a > pm, wa,
                        jnp.where(pm > a, wp, jnp.minimum(wa, wp)))
        hb = mi > ninf
        prev = ainv_v[sl]
        price_v[sl] = jnp.where(hb, price_v[sl] + mi, price_v[sl])
        ainv_v[sl] = jnp.where(hb, win, prev)
        mprev = hb & (prev >= 0)
        plsc.store_scatter(unass_v, [jnp.maximum(prev, 0)],
                           jnp.full((_L,), 1, jnp.int32), mask=mprev)
        plsc.store_scatter(unass_v, [jnp.where(hb, win, 0)],
                           jnp.zeros((_L,), jnp.int32), mask=hb)
        maxinc_v[sl] = jnp.full((_L,), ninf, jnp.float32)
        winner_v[sl] = jnp.full((_L,), n, jnp.int32)
        return c

    def round_body(t, c):
        lax.fori_loop(half * hblk, half * hblk + hblk, phase_a, 0)
        pltpu.sync_copy(maxinc_v, sh_max.at[s])
        pltpu.sync_copy(winner_v, sh_win.at[s])
        plsc.subcore_barrier()
        pltpu.sync_copy(sh_max.at[partner], pmax_v)
        pltpu.sync_copy(sh_win.at[partner], pwin_v)
        plsc.subcore_barrier()
        lax.fori_loop(0, nblk, phase_b, 0)
        return c

    lax.fori_loop(0, iters, round_body, 0)

    def epi(ch, c):
        sl = pl.ds(ch * _L, _L)
        jids = ch * _L + lane
        owners = ainv_v[sl]
        m = owners >= 0
        oc = jnp.maximum(owners, 0)
        d0 = plsc.load_gather(x1p0, [oc]) - x2p0[sl]
        d1 = plsc.load_gather(x1p1, [oc]) - x2p1[sl]
        d2 = plsc.load_gather(x1p2, [oc]) - x2p2[sl]
        dd = (d0 * d0 + d1 * d1) + d2 * d2
        plsc.store_scatter(dist_v, [oc], dd, mask=m)
        plsc.store_scatter(ass_v, [oc], jids, mask=m)
        return c

    @pl.when(half == 0)
    def _():
        lax.fori_loop(0, nblk, epi, 0)
        pltpu.sync_copy(ass_v, ass_hbm.at[b])
        pltpu.sync_copy(dist_v, dist_hbm.at[b])


def kernel(input1, input2, eps, iters):
    b, n, _ = input1.shape
    x1planes = [input1[:, :, k] for k in range(3)]
    x2planes = [input2[:, :, k] for k in range(3)]
    eps_a = jnp.full((_L,), eps, jnp.float32)
    it_a = jnp.full((_L,), iters, jnp.float32)
    mesh = plsc.VectorSubcoreMesh(
        core_axis_name="c", subcore_axis_name="s",
        num_cores=_NC, num_subcores=_NS)
    f = functools.partial(
        pl.kernel,
        out_type=[
            jax.ShapeDtypeStruct((b, _N), jnp.int32),
            jax.ShapeDtypeStruct((b, _N), jnp.float32),
        ],
        mesh=mesh,
        compiler_params=pltpu.CompilerParams(
            use_tc_tiling_on_sc=False, needs_layout_passes=False),
        scratch_types=(
            [pltpu.VMEM((_L,), jnp.float32)]
            + [pltpu.VMEM((_N,), jnp.float32)] * 6
            + [pltpu.VMEM((_N,), jnp.float32),
               pltpu.VMEM((_N,), jnp.int32),
               pltpu.VMEM((_N,), jnp.int32),
               pltpu.VMEM((_N,), jnp.float32),
               pltpu.VMEM((_N,), jnp.int32),
               pltpu.VMEM((_N,), jnp.int32),
               pltpu.VMEM((_N,), jnp.float32),
               pltpu.VMEM((_N,), jnp.float32),
               pltpu.VMEM((_N,), jnp.int32),
               pltpu.VMEM_SHARED((_NS, _N), jnp.float32),
               pltpu.VMEM_SHARED((_NS, _N), jnp.int32)]
        ),
    )(_sc_auction_body)
    ass, dist = f(eps_a, it_a, *x1planes, *x2planes)
    return dist, ass
